# baseline (speedup 1.0000x reference)
import functools

import jax
import jax.numpy as jnp
from jax import lax
from jax.experimental import pallas as pl
from jax.experimental.pallas import tpu as pltpu
from jax.experimental.pallas import tpu_sc as plsc

NC = 2
NS = 16
NW = NC * NS
LANES = 16
NBUF = 3


def _make_kernel(B, S, V, D):
    chunk = S // NW

    mesh = plsc.VectorSubcoreMesh(core_axis_name="c", subcore_axis_name="s")

    @functools.partial(
        pl.kernel,
        out_type=jax.ShapeDtypeStruct((B * S, D), jnp.float32),
        mesh=mesh,
        scratch_types=[
            pltpu.VMEM((chunk, D), jnp.float32),
            pltpu.VMEM((chunk,), jnp.int32),
            pltpu.VMEM((B, chunk), jnp.int32),
            pltpu.VMEM((chunk, D), jnp.float32),
            pltpu.VMEM((chunk, D), jnp.float32),
            pltpu.VMEM((chunk, D), jnp.float32),
            pltpu.SemaphoreType.DMA,
            pltpu.SemaphoreType.DMA,
            pltpu.SemaphoreType.DMA,
            pltpu.SemaphoreType.DMA,
            pltpu.SemaphoreType.DMA,
            pltpu.SemaphoreType.DMA,
            pltpu.SemaphoreType.DMA,
            pltpu.SemaphoreType.DMA,
        ],
    )
    def emb_kernel(ids_hbm, table_hbm, pos_hbm, out_hbm,
                   pos_v, pidx_v, idx_all, rb0, rb1, rb2,
                   psem, isem, g0, g1, g2, o0, o1, o2):
        rows = [rb0, rb1, rb2]
        gsem = [g0, g1, g2]
        osem = [o0, o1, o2]

        cid = lax.axis_index("c")
        sid = lax.axis_index("s")
        wid = sid * NC + cid
        base = wid * chunk

        for j in range(chunk // LANES):
            pidx_v[pl.ds(j * LANES, LANES)] = (
                lax.iota(jnp.int32, LANES) + (base + 1 + j * LANES)
            )
        pltpu.async_copy(pos_hbm.at[pidx_v], pos_v, psem)

        for b in range(B):
            pltpu.async_copy(ids_hbm.at[pl.ds(b * S + base, chunk)],
                             idx_all.at[b], isem)
        for b in range(B):
            pltpu.make_async_copy(ids_hbm.at[pl.ds(b * S + base, chunk)],
                                  idx_all.at[b], isem).wait()
        pltpu.make_async_copy(pos_hbm.at[pidx_v], pos_v, psem).wait()

        def gather_start(b, s):
            pltpu.async_copy(table_hbm.at[idx_all.at[b]], rows[s], gsem[s])

        def gather_wait(b, s):
            pltpu.make_async_copy(table_hbm.at[idx_all.at[b]], rows[s],
                                  gsem[s]).wait()

        def out_start(b, s):
            pltpu.async_copy(rows[s], out_hbm.at[pl.ds(b * S + base, chunk)],
                             osem[s])

        def out_wait(b, s):
            pltpu.make_async_copy(rows[s],
                                  out_hbm.at[pl.ds(b * S + base, chunk)],
                                  osem[s]).wait()

        def step(b, s, issue_next=True, wait_out=True):
            sn = (s + 1) % NBUF
            if issue_next:
                if wait_out:
                    out_wait(b - 2, sn)
                gather_start(b + 1, sn)
            gather_wait(b, s)
            rb = rows[s]

            def row_body(r, c2):
                for cc in range(D // LANES):
                    sl = pl.ds(cc * LANES, LANES)
                    plsc.addupdate(rb.at[r, sl], pos_v[r, sl])
                return c2

            lax.fori_loop(0, chunk, row_body, 0)
            out_start(b, s)

        gather_start(0, 0)
        step(0, 0, wait_out=False)
        step(1, 1, wait_out=False)

        def main_body(i, carry):
            b0 = 2 + NBUF * i
            step(b0, 2)
            step(b0 + 1, 0)
            step(b0 + 2, 1)
            return carry

        lax.fori_loop(0, (B - 5) // NBUF, main_body, 0)
        step(B - 3, 2)
        step(B - 2, 0)
        step(B - 1, 1, issue_next=False)
        out_wait(B - 3, 2)
        out_wait(B - 2, 0)
        out_wait(B - 1, 1)

    return emb_kernel


def kernel(input_ids, charactor_embeddings, position_table):
    B, S = input_ids.shape
    V, D = charactor_embeddings.shape
    fn = _make_kernel(B, S, V, D)
    out = fn(input_ids.reshape(B * S), charactor_embeddings, position_table)
    return out.reshape(B, S, D)


# speedup vs baseline: 1.0170x; 1.0170x over previous
import functools

import jax
import jax.numpy as jnp
from jax import lax
from jax.experimental import pallas as pl
from jax.experimental.pallas import tpu as pltpu
from jax.experimental.pallas import tpu_sc as plsc

NC = 2
NS = 16
NW = NC * NS
LANES = 16
NBUF = 4


def _make_kernel(B, S, V, D):
    chunk = S // NW

    mesh = plsc.VectorSubcoreMesh(core_axis_name="c", subcore_axis_name="s")

    @functools.partial(
        pl.kernel,
        out_type=jax.ShapeDtypeStruct((B * S, D), jnp.float32),
        mesh=mesh,
        scratch_types=[
            pltpu.VMEM((chunk, D), jnp.float32),
            pltpu.VMEM((chunk,), jnp.int32),
            pltpu.VMEM((B, chunk), jnp.int32),
            pltpu.VMEM((chunk, D), jnp.float32),
            pltpu.VMEM((chunk, D), jnp.float32),
            pltpu.VMEM((chunk, D), jnp.float32),
            pltpu.VMEM((chunk, D), jnp.float32),
            pltpu.SemaphoreType.DMA,
            pltpu.SemaphoreType.DMA,
            pltpu.SemaphoreType.DMA,
            pltpu.SemaphoreType.DMA,
            pltpu.SemaphoreType.DMA,
            pltpu.SemaphoreType.DMA,
            pltpu.SemaphoreType.DMA,
            pltpu.SemaphoreType.DMA,
            pltpu.SemaphoreType.DMA,
            pltpu.SemaphoreType.DMA,
        ],
    )
    def emb_kernel(ids_hbm, table_hbm, pos_hbm, out_hbm,
                   pos_v, pidx_v, idx_all, rb0, rb1, rb2, rb3,
                   psem, isem, g0, g1, g2, g3, o0, o1, o2, o3):
        rows = [rb0, rb1, rb2, rb3]
        gsem = [g0, g1, g2, g3]
        osem = [o0, o1, o2, o3]

        cid = lax.axis_index("c")
        sid = lax.axis_index("s")
        wid = sid * NC + cid
        base = wid * chunk

        for j in range(chunk // LANES):
            pidx_v[pl.ds(j * LANES, LANES)] = (
                lax.iota(jnp.int32, LANES) + (base + 1 + j * LANES)
            )
        pltpu.async_copy(pos_hbm.at[pidx_v], pos_v, psem)

        for b in range(B):
            pltpu.async_copy(ids_hbm.at[pl.ds(b * S + base, chunk)],
                             idx_all.at[b], isem)
        for b in range(B):
            pltpu.make_async_copy(ids_hbm.at[pl.ds(b * S + base, chunk)],
                                  idx_all.at[b], isem).wait()
        pltpu.make_async_copy(pos_hbm.at[pidx_v], pos_v, psem).wait()

        def gather_start(b, s):
            pltpu.async_copy(table_hbm.at[idx_all.at[b]], rows[s], gsem[s])

        def gather_wait(b, s):
            pltpu.make_async_copy(table_hbm.at[idx_all.at[b]], rows[s],
                                  gsem[s]).wait()

        def out_start(b, s):
            pltpu.async_copy(rows[s], out_hbm.at[pl.ds(b * S + base, chunk)],
                             osem[s])

        def out_wait(b, s):
            pltpu.make_async_copy(rows[s],
                                  out_hbm.at[pl.ds(b * S + base, chunk)],
                                  osem[s]).wait()

        def step(b, s, issue_next=True, wait_out=True):
            sn = (s + 2) % NBUF
            if issue_next:
                if wait_out:
                    out_wait(b - 2, sn)
                gather_start(b + 2, sn)
            gather_wait(b, s)
            rb = rows[s]

            def row_body(r, c2):
                for cc in range(D // LANES):
                    sl = pl.ds(cc * LANES, LANES)
                    plsc.addupdate(rb.at[r, sl], pos_v[r, sl])
                return c2

            lax.fori_loop(0, chunk, row_body, 0)
            out_start(b, s)

        gather_start(0, 0)
        gather_start(1, 1)
        step(0, 0, wait_out=False)
        step(1, 1, wait_out=False)

        def main_body(i, carry):
            b0 = 2 + NBUF * i
            step(b0, 2)
            step(b0 + 1, 3)
            step(b0 + 2, 0)
            step(b0 + 3, 1)
            return carry

        lax.fori_loop(0, (B - 4) // NBUF, main_body, 0)
        step(B - 2, 2, issue_next=False)
        step(B - 1, 3, issue_next=False)
        out_wait(B - 4, 0)
        out_wait(B - 3, 1)
        out_wait(B - 2, 2)
        out_wait(B - 1, 3)

    return emb_kernel


def kernel(input_ids, charactor_embeddings, position_table):
    B, S = input_ids.shape
    V, D = charactor_embeddings.shape
    fn = _make_kernel(B, S, V, D)
    out = fn(input_ids.reshape(B * S), charactor_embeddings, position_table)
    return out.reshape(B, S, D)
